# speedup vs baseline: 1.1067x; 1.1067x over previous
import jax
import jax.numpy as jnp
from jax.experimental import pallas as pl

_N_DIMS = 1000


def _onehot_block(x_ref, out_ref):
    x = x_ref[...]
    idx = jnp.argmax(x, axis=1).astype(jnp.int32)
    cols = jax.lax.broadcasted_iota(jnp.int32, x.shape, 1)
    out_ref[...] = (cols == idx[:, None]).astype(jnp.float32)


def kernel(x):
    n = x.shape[0]
    blk = 512
    out = pl.pallas_call(
        _onehot_block,
        grid=(n // blk,),
        in_specs=[pl.BlockSpec((blk, _N_DIMS), lambda i: (i, 0))],
        out_specs=pl.BlockSpec((blk, _N_DIMS), lambda i: (i, 0)),
        out_shape=jax.ShapeDtypeStruct((n, _N_DIMS), jnp.float32),
    )(x)
    return out.reshape(n, 1, _N_DIMS)
